# baseline (speedup 1.0000x reference)
import functools

import jax
import jax.numpy as jnp
from jax import lax
from jax.experimental import pallas as pl
from jax.experimental.pallas import tpu as pltpu
from jax.experimental.pallas import tpu_sc as plsc

N_NODES = 100000
D_STRUCT = 131
D_SPATIAL = 64
D_OUT = 256
D_HALF = D_OUT // 2

NC, NS = 2, 16
NW = NC * NS
B_PER_W = 3200
N_IDX = NW * B_PER_W
CHUNK = 40

TC_BLOCK = 512


def _sc_gather_mean(table, idx0, idx1, idx2):
    mesh = plsc.VectorSubcoreMesh(core_axis_name="c", subcore_axis_name="s")

    @functools.partial(
        pl.kernel,
        out_type=jax.ShapeDtypeStruct((N_NODES, D_OUT), jnp.float32),
        mesh=mesh,
        scratch_types=[
            pltpu.VMEM((B_PER_W,), jnp.int32),
            pltpu.VMEM((B_PER_W,), jnp.int32),
            pltpu.VMEM((B_PER_W,), jnp.int32),
            [pltpu.VMEM((CHUNK, D_HALF), jnp.int32) for _ in range(2)],
            [pltpu.VMEM((CHUNK, D_HALF), jnp.int32) for _ in range(2)],
            [pltpu.VMEM((CHUNK, D_HALF), jnp.int32) for _ in range(2)],
            [pltpu.VMEM((CHUNK, D_HALF), jnp.int32) for _ in range(2)],
            [pltpu.VMEM((CHUNK, D_OUT), jnp.float32) for _ in range(2)],
            [pltpu.SemaphoreType.DMA for _ in range(2)],
        ],
    )
    def k(table_hbm, i0_hbm, i1_hbm, i2_hbm, out_hbm,
          i0_v, i1_v, i2_v, sf, g0, g1, g2, acc, sems):
        wid = lax.axis_index("s") * NC + lax.axis_index("c")
        wbase = wid * B_PER_W
        n_chunks = jnp.where(wid == NW - 1, 800 // CHUNK, B_PER_W // CHUNK)
        pltpu.sync_copy(i0_hbm.at[pl.ds(wbase, B_PER_W)], i0_v)
        pltpu.sync_copy(i1_hbm.at[pl.ds(wbase, B_PER_W)], i1_v)
        pltpu.sync_copy(i2_hbm.at[pl.ds(wbase, B_PER_W)], i2_v)

        hi_mask = jnp.full((16,), -65536, dtype=jnp.int32)

        def issue4(c, b):
            lbase = c * CHUNK
            ds = pltpu.async_copy(
                table_hbm.at[pl.ds(wbase + lbase, CHUNK)], sf[b], sems[b])
            dg = [
                pltpu.async_copy(
                    table_hbm.at[iv.at[pl.ds(lbase, CHUNK)]], gk[b], sems[b])
                for gk, iv in zip((g0, g1, g2), (i0_v, i1_v, i2_v))
            ]
            return [ds] + dg

        def compute_and_store(c, b):
            def row_body(j, c2):
                for h in range(D_HALF // 16):
                    sl = pl.ds(h * 16, 16)
                    ws = sf[b][j, sl]
                    w0 = g0[b][j, sl]
                    w1 = g1[b][j, sl]
                    w2 = g2[b][j, sl]

                    def lo(w):
                        return lax.bitcast_convert_type(
                            lax.shift_left(w, 16), jnp.float32)

                    def hi(w):
                        return lax.bitcast_convert_type(
                            lax.bitwise_and(w, hi_mask), jnp.float32)

                    acc[b][j, sl] = (
                        (lo(ws) + lo(w0)) + (lo(w1) + lo(w2))) * 0.25
                    acc[b][j, pl.ds(D_HALF + h * 16, 16)] = (
                        (hi(ws) + hi(w0)) + (hi(w1) + hi(w2))) * 0.25
                return c2

            lax.fori_loop(0, CHUNK, row_body, 0, unroll=2)
            pltpu.sync_copy(acc[b], out_hbm.at[pl.ds(wbase + c * CHUNK, CHUNK)])

        def pair_body(i, carry):
            c0 = 2 * i
            da = issue4(c0, 0)
            db = issue4(c0 + 1, 1)
            for d in da:
                d.wait()
            compute_and_store(c0, 0)
            for d in db:
                d.wait()
            compute_and_store(c0 + 1, 1)
            return carry

        lax.fori_loop(0, n_chunks // 2, pair_body, 0)

    return k(table, idx0, idx1, idx2)


def _p_body(stt_ref, wa_ref, ba_ref, tbl_ref):
    p = (
        lax.dot_general(
            stt_ref[...], wa_ref[...],
            dimension_numbers=(((0,), (0,)), ((), ())),
            preferred_element_type=jnp.float32,
        )
        + ba_ref[...]
    )
    lo = lax.bitcast_convert_type(
        p[:, :D_HALF].astype(jnp.bfloat16), jnp.uint16).astype(jnp.uint32)
    hi = lax.bitcast_convert_type(
        p[:, D_HALF:].astype(jnp.bfloat16), jnp.uint16).astype(jnp.uint32)
    word = jnp.bitwise_or(jnp.left_shift(hi, 16), lo)
    tbl_ref[...] = lax.bitcast_convert_type(word, jnp.int32)


def _tc_p(structural_t, WaT, b_agg):
    grid = (pl.cdiv(N_NODES, TC_BLOCK),)
    full = lambda i: (0, 0)
    return pl.pallas_call(
        _p_body,
        grid=grid,
        in_specs=[
            pl.BlockSpec((D_STRUCT, TC_BLOCK), lambda i: (0, i)),
            pl.BlockSpec((D_STRUCT, D_OUT), full),
            pl.BlockSpec((1, D_OUT), full),
        ],
        out_specs=pl.BlockSpec((TC_BLOCK, D_HALF), lambda i: (i, 0)),
        out_shape=jax.ShapeDtypeStruct((N_NODES, D_HALF), jnp.int32),
    )(structural_t, WaT, b_agg)


def _out1_body(spt_ref, stt_ref, wcs_ref, wct_ref, bc_ref, o1_ref):
    dn = (((0,), (0,)), ((), ()))
    o1_ref[...] = (
        lax.dot_general(spt_ref[...], wcs_ref[...], dimension_numbers=dn,
                        preferred_element_type=jnp.float32)
        + lax.dot_general(stt_ref[...], wct_ref[...], dimension_numbers=dn,
                          preferred_element_type=jnp.float32)
        + bc_ref[...]
    )


def _tc_out1(spatial_t, structural_t, WcSp, WcSt, b_comb):
    grid = (pl.cdiv(N_NODES, TC_BLOCK),)
    full = lambda i: (0, 0)
    return pl.pallas_call(
        _out1_body,
        grid=grid,
        in_specs=[
            pl.BlockSpec((D_SPATIAL, TC_BLOCK), lambda i: (0, i)),
            pl.BlockSpec((D_STRUCT, TC_BLOCK), lambda i: (0, i)),
            pl.BlockSpec((D_SPATIAL, D_OUT), full),
            pl.BlockSpec((D_STRUCT, D_OUT), full),
            pl.BlockSpec((1, D_OUT), full),
        ],
        out_specs=pl.BlockSpec((TC_BLOCK, D_OUT), lambda i: (i, 0)),
        out_shape=jax.ShapeDtypeStruct((N_NODES, D_OUT), jnp.float32),
    )(spatial_t, structural_t, WcSp, WcSt, b_comb)


def kernel(spatial, structural, neighbour, W_comb, b_comb, W_agg, b_agg):
    idx_t = neighbour.astype(jnp.int32).T
    pad = N_IDX - N_NODES
    idx0 = jnp.pad(idx_t[0], (0, pad))
    idx1 = jnp.pad(idx_t[1], (0, pad))
    idx2 = jnp.pad(idx_t[2], (0, pad))

    WcT = W_comb.T
    WcSp = WcT[:D_SPATIAL]
    WcSt = WcT[D_SPATIAL:]
    WaT = W_agg.T
    spatial_t = spatial.T
    structural_t = structural.T

    table = _tc_p(structural_t, WaT, b_agg.reshape(1, D_OUT))
    out2 = _sc_gather_mean(table, idx0, idx1, idx2)
    out1 = _tc_out1(spatial_t, structural_t, WcSp, WcSt,
                    b_comb.reshape(1, D_OUT))
    return (out1, out2)


# speedup vs baseline: 1.1871x; 1.1871x over previous
import functools

import jax
import jax.numpy as jnp
from jax import lax
from jax.experimental import pallas as pl
from jax.experimental.pallas import tpu as pltpu
from jax.experimental.pallas import tpu_sc as plsc

N_NODES = 100000
D_STRUCT = 131
D_SPATIAL = 64
D_OUT = 256

NC, NS = 2, 16
NW = NC * NS
B_PER_W = 3200
N_IDX = NW * B_PER_W
CHUNK = 40
SLICES = D_OUT // 16

TC_BLOCK = 512


def _sc_gather_mean(table, nb_flat):
    mesh = plsc.VectorSubcoreMesh(core_axis_name="c", subcore_axis_name="s")

    @functools.partial(
        pl.kernel,
        out_type=jax.ShapeDtypeStruct((N_NODES, D_OUT), jnp.float32),
        mesh=mesh,
        scratch_types=[
            pltpu.VMEM((B_PER_W,), jnp.int32),
            pltpu.VMEM((B_PER_W,), jnp.int32),
            pltpu.VMEM((B_PER_W,), jnp.int32),
            [pltpu.VMEM((CHUNK, D_OUT), jnp.float32) for _ in range(2)],
            [pltpu.VMEM((CHUNK, D_OUT), jnp.float32) for _ in range(2)],
            [pltpu.VMEM((CHUNK, D_OUT), jnp.float32) for _ in range(2)],
            [pltpu.VMEM((CHUNK, D_OUT), jnp.float32) for _ in range(2)],
            [pltpu.SemaphoreType.DMA for _ in range(2)],
            [pltpu.SemaphoreType.DMA for _ in range(2)],
        ],
    )
    def k(table_hbm, nb_hbm, out_hbm,
          i0_v, i1_v, i2_v, g0, g1, g2, acc, sems, osems):
        wid = lax.axis_index("s") * NC + lax.axis_index("c")
        wbase = wid * B_PER_W
        last = wid == NW - 1
        n_chunks = jnp.where(last, 800 // CHUNK, B_PER_W // CHUNK)

        @pl.when(jnp.logical_not(last))
        def _():
            for kk, iv in enumerate((i0_v, i1_v, i2_v)):
                pltpu.sync_copy(
                    nb_hbm.at[pl.ds(kk * N_NODES + wbase, B_PER_W)], iv)

        @pl.when(last)
        def _():
            for kk, iv in enumerate((i0_v, i1_v, i2_v)):
                pltpu.sync_copy(
                    nb_hbm.at[pl.ds(kk * N_NODES + wbase, 800)],
                    iv.at[pl.ds(0, 800)])

        def issue4(c, b):
            lbase = c * CHUNK
            ds = pltpu.async_copy(
                table_hbm.at[pl.ds(wbase + lbase, CHUNK)], acc[b], sems[b])
            dg = [
                pltpu.async_copy(
                    table_hbm.at[iv.at[pl.ds(lbase, CHUNK)]], gk[b], sems[b])
                for gk, iv in zip((g0, g1, g2), (i0_v, i1_v, i2_v))
            ]
            return [ds] + dg

        def compute_and_store(c, b):
            def row_body(j, c2):
                for d in range(SLICES):
                    sl = pl.ds(d * 16, 16)
                    acc[b][j, sl] = (
                        acc[b][j, sl] + g0[b][j, sl] + g1[b][j, sl]
                        + g2[b][j, sl]
                    ) * 0.25
                return c2

            lax.fori_loop(0, CHUNK, row_body, 0, unroll=2)
            pltpu.async_copy(
                acc[b], out_hbm.at[pl.ds(wbase + c * CHUNK, CHUNK)], osems[b])

        def wait_out(c, b):
            pltpu.make_async_copy(
                acc[b], out_hbm.at[pl.ds(wbase + c * CHUNK, CHUNK)],
                osems[b]).wait()

        def pair_body(i, carry):
            c0 = 2 * i

            @pl.when(c0 >= 2)
            def _():
                wait_out(c0 - 2, 0)

            da = issue4(c0, 0)

            @pl.when(c0 >= 2)
            def _():
                wait_out(c0 - 1, 1)

            db = issue4(c0 + 1, 1)
            for d in da:
                d.wait()
            compute_and_store(c0, 0)
            for d in db:
                d.wait()
            compute_and_store(c0 + 1, 1)
            return carry

        lax.fori_loop(0, n_chunks // 2, pair_body, 0)
        wait_out(n_chunks - 2, 0)
        wait_out(n_chunks - 1, 1)

    return k(table, nb_flat)


def _p_body(stt_ref, wa_ref, ba_ref, p_ref):
    p_ref[...] = (
        lax.dot_general(
            stt_ref[...], wa_ref[...],
            dimension_numbers=(((0,), (0,)), ((), ())),
            preferred_element_type=jnp.float32,
        )
        + ba_ref[...]
    )


def _tc_p(structural_t, WaT, b_agg):
    grid = (pl.cdiv(N_NODES, TC_BLOCK),)
    full = lambda i: (0, 0)
    return pl.pallas_call(
        _p_body,
        grid=grid,
        in_specs=[
            pl.BlockSpec((D_STRUCT, TC_BLOCK), lambda i: (0, i)),
            pl.BlockSpec((D_STRUCT, D_OUT), full),
            pl.BlockSpec((1, D_OUT), full),
        ],
        out_specs=pl.BlockSpec((TC_BLOCK, D_OUT), lambda i: (i, 0)),
        out_shape=jax.ShapeDtypeStruct((N_NODES, D_OUT), jnp.float32),
    )(structural_t, WaT, b_agg)


def _out1_body(spt_ref, stt_ref, wc_ref, bc_ref, o1_ref):
    dn = (((0,), (0,)), ((), ()))
    wc = wc_ref[...]
    o1_ref[...] = (
        lax.dot_general(spt_ref[...], wc[:D_SPATIAL], dimension_numbers=dn,
                        preferred_element_type=jnp.float32)
        + lax.dot_general(stt_ref[...], wc[D_SPATIAL:], dimension_numbers=dn,
                          preferred_element_type=jnp.float32)
        + bc_ref[...]
    )


def _tc_out1(spatial_t, structural_t, WcT, b_comb):
    grid = (pl.cdiv(N_NODES, TC_BLOCK),)
    full = lambda i: (0, 0)
    return pl.pallas_call(
        _out1_body,
        grid=grid,
        in_specs=[
            pl.BlockSpec((D_SPATIAL, TC_BLOCK), lambda i: (0, i)),
            pl.BlockSpec((D_STRUCT, TC_BLOCK), lambda i: (0, i)),
            pl.BlockSpec((D_SPATIAL + D_STRUCT, D_OUT), full),
            pl.BlockSpec((1, D_OUT), full),
        ],
        out_specs=pl.BlockSpec((TC_BLOCK, D_OUT), lambda i: (i, 0)),
        out_shape=jax.ShapeDtypeStruct((N_NODES, D_OUT), jnp.float32),
    )(spatial_t, structural_t, WcT, b_comb)


def kernel(spatial, structural, neighbour, W_comb, b_comb, W_agg, b_agg):
    nb_flat = neighbour.astype(jnp.int32).T.reshape(-1)

    WcT = W_comb.T
    WaT = W_agg.T
    spatial_t = spatial.T
    structural_t = structural.T

    P = _tc_p(structural_t, WaT, b_agg.reshape(1, D_OUT))
    out2 = _sc_gather_mean(P, nb_flat)
    out1 = _tc_out1(spatial_t, structural_t, WcT, b_comb.reshape(1, D_OUT))
    return (out1, out2)
